# baseline (speedup 1.0000x reference)
import jax
import jax.numpy as jnp
from jax import lax
from jax.experimental import pallas as pl
from jax.experimental.pallas import tpu as pltpu
from jax.experimental.pallas import tpu_sc as plsc

VOCAB = 256
L = 2097152

_NC = 2
_NS = 16
_NW = _NC * _NS
_CH = L // _NW
_GB = 128
_NG = 8
_SB = _GB * _NG
_NSB = _CH // _SB
_HALO = 16


def _body(tbl_hbm, x_hbm, out_hbm, xv, idx, gout, gsem):
    wid = lax.axis_index("s") * _NC + lax.axis_index("c")
    base = wid * _CH

    pltpu.sync_copy(x_hbm.at[pl.ds(base, _CH)], xv.at[pl.ds(_HALO, _CH)])

    @pl.when(wid > 0)
    def _():
        pltpu.sync_copy(x_hbm.at[pl.ds(base - _HALO, _HALO)],
                        xv.at[pl.ds(0, _HALO)])

    @pl.when(wid == 0)
    def _():
        v0 = xv[pl.ds(_HALO, 16)]
        xv[pl.ds(0, 16)] = jnp.full((16,), 0, jnp.int32) + v0[0]

    def step(sb, _):
        def grpfn(g, _):
            o = sb * _SB + g * 16
            a = xv[pl.ds(_HALO - 2 + o, 16)]
            b = xv[pl.ds(_HALO - 1 + o, 16)]
            c = xv[pl.ds(_HALO + o, 16)]
            idx[g // 8, pl.ds((g % 8) * 16, 16)] = a * 65536 + b * 256 + c
            return 0
        lax.fori_loop(0, _SB // 16, grpfn, 0)

        for j in range(_NG):
            pltpu.make_async_copy(tbl_hbm.at[idx.at[j]],
                                  gout.at[pl.ds(j * _GB, _GB)],
                                  gsem).start()
        for j in range(_NG):
            pltpu.make_async_copy(tbl_hbm.at[idx.at[j]],
                                  gout.at[pl.ds(j * _GB, _GB)],
                                  gsem).wait()

        pltpu.sync_copy(gout, out_hbm.at[pl.ds(base + sb * _SB, _SB)])
        return 0

    lax.fori_loop(0, _NSB, step, 0)


@jax.jit
def _ngram_lookup(tbl, x):
    mesh = plsc.VectorSubcoreMesh(core_axis_name="c", subcore_axis_name="s")
    return pl.kernel(
        _body,
        out_type=jax.ShapeDtypeStruct((L,), jnp.float32),
        mesh=mesh,
        scratch_types=[
            pltpu.VMEM((_HALO + _CH,), jnp.int32),
            pltpu.VMEM((_NG, _GB), jnp.int32),
            pltpu.VMEM((_SB,), jnp.float32),
            pltpu.SemaphoreType.DMA,
        ],
    )(tbl, x)


def kernel(probs, x):
    return _ngram_lookup(probs.reshape(-1), x)


# speedup vs baseline: 1.1244x; 1.1244x over previous
import jax
import jax.numpy as jnp
from jax import lax
from jax.experimental import pallas as pl
from jax.experimental.pallas import tpu as pltpu
from jax.experimental.pallas import tpu_sc as plsc

VOCAB = 256
L = 2097152

_NC = 2
_NS = 16
_NW = _NC * _NS
_CH = L // _NW
_GB = 128
_NG = 8
_SB = _GB * _NG
_NSB = _CH // _SB
_HALO = 16


def _body(tbl_hbm, x_hbm, out_hbm, xv, idx0, idx1, gout0, gout1,
          gsem0, gsem1, osem0, osem1):
    wid = lax.axis_index("s") * _NC + lax.axis_index("c")
    base = wid * _CH
    idx_v = (idx0, idx1)
    gout = (gout0, gout1)
    gsem = (gsem0, gsem1)
    osem = (osem0, osem1)

    pltpu.sync_copy(x_hbm.at[pl.ds(base, _CH)], xv.at[pl.ds(_HALO, _CH)])

    @pl.when(wid > 0)
    def _():
        pltpu.sync_copy(x_hbm.at[pl.ds(base - _HALO, _HALO)],
                        xv.at[pl.ds(0, _HALO)])

    @pl.when(wid == 0)
    def _():
        v0 = xv[pl.ds(_HALO, 16)]
        xv[pl.ds(0, 16)] = jnp.full((16,), 0, jnp.int32) + v0[0]

    def compute_idx(sb, slot):
        def grpfn(q, _):
            for u in range(4):
                g = q * 4 + u
                o = sb * _SB + g * 16
                a = xv[pl.ds(_HALO - 2 + o, 16)]
                b = xv[pl.ds(_HALO - 1 + o, 16)]
                c = xv[pl.ds(_HALO + o, 16)]
                idx_v[slot][g // 8, pl.ds((g % 8) * 16, 16)] = (
                    (a << 16) | (b << 8) | c)
            return 0
        lax.fori_loop(0, _SB // 64, grpfn, 0)

    def gather(j, slot):
        return pltpu.make_async_copy(tbl_hbm.at[idx_v[slot].at[j]],
                                     gout[slot].at[pl.ds(j * _GB, _GB)],
                                     gsem[slot])

    def store(sb, slot):
        return pltpu.make_async_copy(gout[slot],
                                     out_hbm.at[pl.ds(base + sb * _SB, _SB)],
                                     osem[slot])

    def step(sb, slot):
        @pl.when(sb >= 2)
        def _():
            store(sb - 2, slot).wait()

        for j in range(_NG):
            gather(j, slot).start()

        @pl.when(sb < _NSB - 1)
        def _():
            compute_idx(sb + 1, 1 - slot)

        for j in range(_NG):
            gather(j, slot).wait()
        store(sb, slot).start()

    compute_idx(0, 0)

    def pair(p, _):
        step(2 * p, 0)
        step(2 * p + 1, 1)
        return 0

    lax.fori_loop(0, _NSB // 2, pair, 0)
    store(_NSB - 2, 0).wait()
    store(_NSB - 1, 1).wait()


@jax.jit
def _ngram_lookup(tbl, x):
    mesh = plsc.VectorSubcoreMesh(core_axis_name="c", subcore_axis_name="s")
    return pl.kernel(
        _body,
        out_type=jax.ShapeDtypeStruct((L,), jnp.float32),
        mesh=mesh,
        scratch_types=[
            pltpu.VMEM((_HALO + _CH,), jnp.int32),
            pltpu.VMEM((_NG, _GB), jnp.int32),
            pltpu.VMEM((_NG, _GB), jnp.int32),
            pltpu.VMEM((_SB,), jnp.float32),
            pltpu.VMEM((_SB,), jnp.float32),
            pltpu.SemaphoreType.DMA,
            pltpu.SemaphoreType.DMA,
            pltpu.SemaphoreType.DMA,
            pltpu.SemaphoreType.DMA,
        ],
    )(tbl, x)


def kernel(probs, x):
    return _ngram_lookup(probs.reshape(-1), x)


# speedup vs baseline: 1.3077x; 1.1631x over previous
import jax
import jax.numpy as jnp
from jax import lax
from jax.experimental import pallas as pl
from jax.experimental.pallas import tpu as pltpu
from jax.experimental.pallas import tpu_sc as plsc

VOCAB = 256
L = 2097152

_NC = 2
_NS = 16
_NW = _NC * _NS
_CH = L // _NW
_GB = 128
_NG = 4
_SB = _GB * _NG
_NSB = _CH // _SB
_NSL = 4
_HALO = 16


def _body(tbl_hbm, x_hbm, out_hbm, xv, *bufs):
    idx_v = bufs[0:4]
    gout = bufs[4:8]
    gsem = bufs[8:12]
    osem = bufs[12:16]
    wid = lax.axis_index("s") * _NC + lax.axis_index("c")
    base = wid * _CH

    pltpu.sync_copy(x_hbm.at[pl.ds(base, _CH)], xv.at[pl.ds(_HALO, _CH)])

    @pl.when(wid > 0)
    def _():
        pltpu.sync_copy(x_hbm.at[pl.ds(base - _HALO, _HALO)],
                        xv.at[pl.ds(0, _HALO)])

    @pl.when(wid == 0)
    def _():
        v0 = xv[pl.ds(_HALO, 16)]
        xv[pl.ds(0, 16)] = jnp.full((16,), 0, jnp.int32) + v0[0]

    def compute_idx(sb, slot):
        def grpfn(q, _):
            for u in range(4):
                g = q * 4 + u
                o = sb * _SB + g * 16
                a = xv[pl.ds(_HALO - 2 + o, 16)]
                b = xv[pl.ds(_HALO - 1 + o, 16)]
                c = xv[pl.ds(_HALO + o, 16)]
                idx_v[slot][g // 8, pl.ds((g % 8) * 16, 16)] = (
                    (a << 16) | (b << 8) | c)
            return 0
        lax.fori_loop(0, _SB // 64, grpfn, 0)

    def gather(j, slot):
        return pltpu.make_async_copy(tbl_hbm.at[idx_v[slot].at[j]],
                                     gout[slot].at[pl.ds(j * _GB, _GB)],
                                     gsem[slot])

    def store(sb, slot):
        return pltpu.make_async_copy(gout[slot],
                                     out_hbm.at[pl.ds(base + sb * _SB, _SB)],
                                     osem[slot])

    def step(sb, slot):
        pslot = (slot - 2) % _NSL

        @pl.when(sb >= 2)
        def _():
            for j in range(_NG):
                gather(j, pslot).wait()
            store(sb - 2, pslot).start()

        @pl.when(sb >= _NSL)
        def _():
            store(sb - _NSL, slot).wait()

        for j in range(_NG):
            gather(j, slot).start()

        @pl.when(sb < _NSB - 1)
        def _():
            compute_idx(sb + 1, (slot + 1) % _NSL)

    compute_idx(0, 0)

    def quad(p, _):
        for u in range(_NSL):
            step(_NSL * p + u, u)
        return 0

    lax.fori_loop(0, _NSB // _NSL, quad, 0)

    for sb in (_NSB - 2, _NSB - 1):
        slot = sb % _NSL
        for j in range(_NG):
            gather(j, slot).wait()
        store(sb, slot).start()
    for sb in range(_NSB - _NSL, _NSB):
        store(sb, sb % _NSL).wait()


@jax.jit
def _ngram_lookup(tbl, x):
    mesh = plsc.VectorSubcoreMesh(core_axis_name="c", subcore_axis_name="s")
    return pl.kernel(
        _body,
        out_type=jax.ShapeDtypeStruct((L,), jnp.float32),
        mesh=mesh,
        scratch_types=[pltpu.VMEM((_HALO + _CH,), jnp.int32)]
        + [pltpu.VMEM((_NG, _GB), jnp.int32) for _ in range(_NSL)]
        + [pltpu.VMEM((_SB,), jnp.float32) for _ in range(_NSL)]
        + [pltpu.SemaphoreType.DMA for _ in range(2 * _NSL)],
    )(tbl, x)


def kernel(probs, x):
    return _ngram_lookup(probs.reshape(-1), x)


# speedup vs baseline: 1.4526x; 1.1108x over previous
import jax
import jax.numpy as jnp
from jax import lax
from jax.experimental import pallas as pl
from jax.experimental.pallas import tpu as pltpu
from jax.experimental.pallas import tpu_sc as plsc

VOCAB = 256
L = 2097152

_NC = 2
_NS = 16
_NW = _NC * _NS
_CH = L // _NW
_GB = 128
_NG = 8
_SB = _GB * _NG
_NSB = _CH // _SB
_NSL = 4
_HALO = 16


def _body(tbl_hbm, x_hbm, out_hbm, xv, *bufs):
    idx_v = bufs[0:4]
    gout = bufs[4:8]
    gsem = bufs[8:12]
    osem = bufs[12:16]
    wid = lax.axis_index("s") * _NC + lax.axis_index("c")
    base = wid * _CH

    pltpu.sync_copy(x_hbm.at[pl.ds(base, _CH)], xv.at[pl.ds(_HALO, _CH)])

    @pl.when(wid > 0)
    def _():
        pltpu.sync_copy(x_hbm.at[pl.ds(base - _HALO, _HALO)],
                        xv.at[pl.ds(0, _HALO)])

    @pl.when(wid == 0)
    def _():
        v0 = xv[pl.ds(_HALO, 16)]
        xv[pl.ds(0, 16)] = jnp.full((16,), 0, jnp.int32) + v0[0]

    def compute_idx(sb, slot):
        def grpfn(q, _):
            for u in range(4):
                g = q * 4 + u
                o = sb * _SB + g * 16
                a = xv[pl.ds(_HALO - 2 + o, 16)]
                b = xv[pl.ds(_HALO - 1 + o, 16)]
                c = xv[pl.ds(_HALO + o, 16)]
                idx_v[slot][g // 8, pl.ds((g % 8) * 16, 16)] = (
                    (a << 16) | (b << 8) | c)
            return 0
        lax.fori_loop(0, _SB // 64, grpfn, 0)

    def gather(j, slot):
        return pltpu.make_async_copy(tbl_hbm.at[idx_v[slot].at[j]],
                                     gout[slot].at[pl.ds(j * _GB, _GB)],
                                     gsem[slot])

    def store(sb, slot):
        return pltpu.make_async_copy(gout[slot],
                                     out_hbm.at[pl.ds(base + sb * _SB, _SB)],
                                     osem[slot])

    def step(sb, slot):
        pslot = (slot - 2) % _NSL

        @pl.when(sb >= 2)
        def _():
            for j in range(_NG):
                gather(j, pslot).wait()
            store(sb - 2, pslot).start()

        @pl.when(sb >= _NSL)
        def _():
            store(sb - _NSL, slot).wait()

        for j in range(_NG):
            gather(j, slot).start()

        @pl.when(sb < _NSB - 1)
        def _():
            compute_idx(sb + 1, (slot + 1) % _NSL)

    compute_idx(0, 0)

    def quad(p, _):
        for u in range(_NSL):
            step(_NSL * p + u, u)
        return 0

    lax.fori_loop(0, _NSB // _NSL, quad, 0)

    for sb in (_NSB - 2, _NSB - 1):
        slot = sb % _NSL
        for j in range(_NG):
            gather(j, slot).wait()
        store(sb, slot).start()
    for sb in range(_NSB - _NSL, _NSB):
        store(sb, sb % _NSL).wait()


@jax.jit
def _ngram_lookup(tbl, x):
    mesh = plsc.VectorSubcoreMesh(core_axis_name="c", subcore_axis_name="s")
    return pl.kernel(
        _body,
        out_type=jax.ShapeDtypeStruct((L,), jnp.float32),
        mesh=mesh,
        scratch_types=[pltpu.VMEM((_HALO + _CH,), jnp.int32)]
        + [pltpu.VMEM((_NG, _GB), jnp.int32) for _ in range(_NSL)]
        + [pltpu.VMEM((_SB,), jnp.float32) for _ in range(_NSL)]
        + [pltpu.SemaphoreType.DMA for _ in range(2 * _NSL)],
    )(tbl, x)


def kernel(probs, x):
    return _ngram_lookup(probs.reshape(-1), x)


# speedup vs baseline: 1.4628x; 1.0070x over previous
import jax
import jax.numpy as jnp
from jax import lax
from jax.experimental import pallas as pl
from jax.experimental.pallas import tpu as pltpu
from jax.experimental.pallas import tpu_sc as plsc

VOCAB = 256
L = 2097152

_NC = 2
_NS = 16
_NW = _NC * _NS
_CH = L // _NW
_GB = 128
_NG = 16
_SB = _GB * _NG
_NSB = _CH // _SB
_NSL = 4
_HALO = 16


def _body(tbl_hbm, x_hbm, out_hbm, xv, *bufs):
    idx_v = bufs[0:4]
    gout = bufs[4:8]
    gsem = bufs[8:12]
    osem = bufs[12:16]
    wid = lax.axis_index("s") * _NC + lax.axis_index("c")
    base = wid * _CH

    pltpu.sync_copy(x_hbm.at[pl.ds(base, _CH)], xv.at[pl.ds(_HALO, _CH)])

    @pl.when(wid > 0)
    def _():
        pltpu.sync_copy(x_hbm.at[pl.ds(base - _HALO, _HALO)],
                        xv.at[pl.ds(0, _HALO)])

    @pl.when(wid == 0)
    def _():
        v0 = xv[pl.ds(_HALO, 16)]
        xv[pl.ds(0, 16)] = jnp.full((16,), 0, jnp.int32) + v0[0]

    def compute_idx(sb, slot):
        def grpfn(q, _):
            for u in range(4):
                g = q * 4 + u
                o = sb * _SB + g * 16
                a = xv[pl.ds(_HALO - 2 + o, 16)]
                b = xv[pl.ds(_HALO - 1 + o, 16)]
                c = xv[pl.ds(_HALO + o, 16)]
                idx_v[slot][g // 8, pl.ds((g % 8) * 16, 16)] = (
                    (a << 16) | (b << 8) | c)
            return 0
        lax.fori_loop(0, _SB // 64, grpfn, 0)

    def gather(j, slot):
        return pltpu.make_async_copy(tbl_hbm.at[idx_v[slot].at[j]],
                                     gout[slot].at[pl.ds(j * _GB, _GB)],
                                     gsem[slot])

    def store(sb, slot):
        return pltpu.make_async_copy(gout[slot],
                                     out_hbm.at[pl.ds(base + sb * _SB, _SB)],
                                     osem[slot])

    def step(sb, slot):
        pslot = (slot - 2) % _NSL

        @pl.when(sb >= 2)
        def _():
            for j in range(_NG):
                gather(j, pslot).wait()
            store(sb - 2, pslot).start()

        @pl.when(sb >= _NSL)
        def _():
            store(sb - _NSL, slot).wait()

        for j in range(_NG):
            gather(j, slot).start()

        @pl.when(sb < _NSB - 1)
        def _():
            compute_idx(sb + 1, (slot + 1) % _NSL)

    compute_idx(0, 0)

    def quad(p, _):
        for u in range(_NSL):
            step(_NSL * p + u, u)
        return 0

    lax.fori_loop(0, _NSB // _NSL, quad, 0)

    for sb in (_NSB - 2, _NSB - 1):
        slot = sb % _NSL
        for j in range(_NG):
            gather(j, slot).wait()
        store(sb, slot).start()
    for sb in range(_NSB - _NSL, _NSB):
        store(sb, sb % _NSL).wait()


@jax.jit
def _ngram_lookup(tbl, x):
    mesh = plsc.VectorSubcoreMesh(core_axis_name="c", subcore_axis_name="s")
    return pl.kernel(
        _body,
        out_type=jax.ShapeDtypeStruct((L,), jnp.float32),
        mesh=mesh,
        scratch_types=[pltpu.VMEM((_HALO + _CH,), jnp.int32)]
        + [pltpu.VMEM((_NG, _GB), jnp.int32) for _ in range(_NSL)]
        + [pltpu.VMEM((_SB,), jnp.float32) for _ in range(_NSL)]
        + [pltpu.SemaphoreType.DMA for _ in range(2 * _NSL)],
    )(tbl, x)


def kernel(probs, x):
    return _ngram_lookup(probs.reshape(-1), x)


# speedup vs baseline: 2.1668x; 1.4812x over previous
import jax
import jax.numpy as jnp
from jax import lax
from jax.experimental import pallas as pl
from jax.experimental.pallas import tpu as pltpu
from jax.experimental.pallas import tpu_sc as plsc

VOCAB = 256
L = 2097152

_NC = 2
_NS = 16
_NW = _NC * _NS
_CH = L // _NW
_GB = 128
_NG = 16
_SB = _GB * _NG
_NSB = _CH // _SB
_NSL = 4
_HALO = 16


def _body(tbl_hbm, x_hbm, out_hbm, xv, *bufs):
    idx_v = bufs[0:4]
    gout = bufs[4:8]
    gsem = bufs[8:12]
    osem = bufs[12:16]
    wid = lax.axis_index("s") * _NC + lax.axis_index("c")
    base = wid * _CH

    pltpu.sync_copy(x_hbm.at[pl.ds(base, _CH)], xv.at[pl.ds(_HALO, _CH)])

    @pl.when(wid > 0)
    def _():
        pltpu.sync_copy(x_hbm.at[pl.ds(base - _HALO, _HALO)],
                        xv.at[pl.ds(0, _HALO)])

    @pl.when(wid == 0)
    def _():
        v0 = xv[pl.ds(_HALO, 16)]
        xv[pl.ds(0, 16)] = jnp.full((16,), 0, jnp.int32) + v0[0]

    def compute_idx(sb, slot):
        def grpfn(q, _):
            for u in range(4):
                g = q * 4 + u
                o = sb * _SB + g * 16
                a = xv[pl.ds(_HALO - 2 + o, 16)]
                b = xv[pl.ds(_HALO - 1 + o, 16)]
                c = xv[pl.ds(_HALO + o, 16)]
                idx_v[slot][g // 8, pl.ds((g % 8) * 16, 16)] = (
                    (a << 16) | ((b >> 3) << 11) | ((c >> 7) << 10)
                    | ((b & 7) << 7) | (c & 127))
            return 0
        lax.fori_loop(0, _SB // 64, grpfn, 0)

    def gather(j, slot):
        return pltpu.make_async_copy(tbl_hbm.at[idx_v[slot].at[j]],
                                     gout[slot].at[pl.ds(j * _GB, _GB)],
                                     gsem[slot])

    def store(sb, slot):
        return pltpu.make_async_copy(gout[slot],
                                     out_hbm.at[pl.ds(base + sb * _SB, _SB)],
                                     osem[slot])

    def step(sb, slot):
        pslot = (slot - 2) % _NSL

        @pl.when(sb >= 2)
        def _():
            for j in range(_NG):
                gather(j, pslot).wait()
            store(sb - 2, pslot).start()

        @pl.when(sb >= _NSL)
        def _():
            store(sb - _NSL, slot).wait()

        for j in range(_NG):
            gather(j, slot).start()

        @pl.when(sb < _NSB - 1)
        def _():
            compute_idx(sb + 1, (slot + 1) % _NSL)

    compute_idx(0, 0)

    def quad(p, _):
        for u in range(_NSL):
            step(_NSL * p + u, u)
        return 0

    lax.fori_loop(0, _NSB // _NSL, quad, 0)

    for sb in (_NSB - 2, _NSB - 1):
        slot = sb % _NSL
        for j in range(_NG):
            gather(j, slot).wait()
        store(sb, slot).start()
    for sb in range(_NSB - _NSL, _NSB):
        store(sb, sb % _NSL).wait()


@jax.jit
def _ngram_lookup(tbl, x):
    mesh = plsc.VectorSubcoreMesh(core_axis_name="c", subcore_axis_name="s")
    return pl.kernel(
        _body,
        out_type=jax.ShapeDtypeStruct((L,), jnp.float32),
        mesh=mesh,
        scratch_types=[pltpu.VMEM((_HALO + _CH,), jnp.int32)]
        + [pltpu.VMEM((_NG, _GB), jnp.int32) for _ in range(_NSL)]
        + [pltpu.VMEM((_SB,), jnp.float32) for _ in range(_NSL)]
        + [pltpu.SemaphoreType.DMA for _ in range(2 * _NSL)],
    )(tbl, x)


def kernel(probs, x):
    tbl = probs.reshape(VOCAB, 32, 8, 2, 128).transpose(0, 1, 3, 2, 4)
    return _ngram_lookup(tbl.reshape(-1), x)
